# baseline (speedup 1.0000x reference)
import functools

import jax
import jax.numpy as jnp
from jax import lax
from jax.experimental import pallas as pl
from jax.experimental.pallas import tpu as pltpu
from jax.experimental.pallas import tpu_sc as plsc

N = 10000
E = 320000
F_IN = 128
HID = 64
HEADS = 8
NCLS = 10
NG = 64

EA = E + N
K = 128
TILES = 16
CORES = 2
EAP = -(-EA // (CORES * TILES * K * 2)) * (CORES * TILES * K * 2)
NP = 10240
ROWS_PT = NP // TILES
K1 = 64
NP2 = 10112
RPT1 = NP2 // TILES
RT = 1000
RTM = 1024


def _sc_mesh():
    return plsc.VectorSubcoreMesh(
        core_axis_name="c", subcore_axis_name="s",
        num_cores=CORES, num_subcores=TILES)


def _pre1_body(x_ref, w_ref, as_ref, ad_ref, xl_ref, asd_ref, add_ref):
    xb = jnp.dot(x_ref[...], w_ref[...], preferred_element_type=jnp.float32)
    xl_ref[...] = xb
    a_s = (xb * as_ref[...]).reshape(RT, HEADS, HID).sum(-1)
    a_d = (xb * ad_ref[...]).reshape(RT, HEADS, HID).sum(-1)
    asd_ref[...] = jnp.concatenate([a_s, a_s], axis=1)
    add_ref[...] = jnp.concatenate([a_d, a_d], axis=1)


def _pre1(x, W1, as1, ad1):
    grid = (N // RT,)
    return pl.pallas_call(
        _pre1_body,
        grid=grid,
        in_specs=[
            pl.BlockSpec((RT, F_IN), lambda i: (i, 0)),
            pl.BlockSpec((F_IN, HEADS * HID), lambda i: (0, 0)),
            pl.BlockSpec((1, HEADS * HID), lambda i: (0, 0)),
            pl.BlockSpec((1, HEADS * HID), lambda i: (0, 0)),
        ],
        out_specs=[
            pl.BlockSpec((RT, HEADS * HID), lambda i: (i, 0)),
            pl.BlockSpec((RT, 16), lambda i: (i, 0)),
            pl.BlockSpec((RT, 16), lambda i: (i, 0)),
        ],
        out_shape=[
            jax.ShapeDtypeStruct((N, HEADS * HID), jnp.float32),
            jax.ShapeDtypeStruct((N, 16), jnp.float32),
            jax.ShapeDtypeStruct((N, 16), jnp.float32),
        ],
    )(x, W1, as1, ad1)


def _l1_body(src_hbm, dst_hbm, asd_hbm, add_hbm, xlc_hbm, z128_hbm, zd_hbm,
             num_out, den_out,
             sidxA, didxA, didxSA, sbufA, dbufA, wbufA, rowsA, srowsA,
             sidxB, didxB, didxSB, sbufB, dbufB, wbufB, rowsB, srowsB,
             num_sh, den_sh, semgA, semgB, semsA, semsB):
    c = lax.axis_index("c")
    s = lax.axis_index("s")
    r0 = s * RPT1
    ept = EAP // TILES
    nblk = ept // K1
    ebase = s * ept
    bufA = (sidxA, didxA, didxSA, sbufA, dbufA, wbufA, rowsA, srowsA,
            semgA, semsA)
    bufB = (sidxB, didxB, didxSB, sbufB, dbufB, wbufB, rowsB, srowsB,
            semgB, semsB)

    for cs in range(CORES):
        @pl.when(c == cs)
        def _core_branch(cs=cs):
            for ckl in range(2):
                ck = cs * 2 + ckl
                h0 = 2 * ck
                h1 = 2 * ck + 1
                first = (cs == 0 and ckl == 0)
                pltpu.sync_copy(z128_hbm.at[pl.ds(r0, RPT1)],
                                num_sh.at[pl.ds(r0, RPT1)])
                if first:
                    pltpu.sync_copy(zd_hbm.at[pl.ds(r0, RPT1)],
                                    den_sh.at[pl.ds(r0, RPT1)])
                plsc.subcore_barrier()

                def pregather(e0, buf, ck=ck):
                    sidx, didx, didxS, sbuf, dbuf, wbuf, rows, srows, \
                        semg, sems = buf
                    pltpu.sync_copy(src_hbm.at[pl.ds(e0, K1)], sidx)
                    pltpu.sync_copy(dst_hbm.at[pl.ds(e0, K1)], didx)
                    pltpu.async_copy(asd_hbm.at[sidx], sbuf, semg)
                    pltpu.async_copy(add_hbm.at[didx], dbuf, semg)
                    pltpu.async_copy(xlc_hbm.at[ck].at[sidx], rows, semg)

                def stage(j, buf, first=first, h0=h0, h1=h1):
                    sidx, didx, didxS, sbuf, dbuf, wbuf, rows, srows, \
                        semg, sems = buf
                    pltpu.make_async_copy(asd_hbm.at[sidx], sbuf, semg).wait()
                    pltpu.make_async_copy(add_hbm.at[didx], dbuf, semg).wait()
                    pltpu.make_async_copy(
                        xlc_hbm.at[0].at[sidx], rows, semg).wait()

                    @pl.when(j > 0)
                    def _drain_prev_scatter():
                        pltpu.make_async_copy(
                            srows, num_sh.at[didxS], sems).wait()
                        if first:
                            pltpu.make_async_copy(
                                wbuf, den_sh.at[didxS], sems).wait()

                    for t in range(K1 // 16):
                        didxS[pl.ds(t * 16, 16)] = didx[pl.ds(t * 16, 16)]

                    def edge(k, _):
                        al = sbuf[k, :] + dbuf[k, :]
                        al = jnp.maximum(al, 0.2 * al)
                        w = jnp.exp(al)
                        if first:
                            wbuf[k, :] = w
                        w0 = w[h0]
                        w1 = w[h1]
                        for t in range(4):
                            srows[k, pl.ds(t * 16, 16)] = (
                                rows[k, pl.ds(t * 16, 16)] * w0)
                        for t in range(4, 8):
                            srows[k, pl.ds(t * 16, 16)] = (
                                rows[k, pl.ds(t * 16, 16)] * w1)
                        return 0

                    lax.fori_loop(0, K1, edge, 0)
                    pltpu.async_copy(srows, num_sh.at[didxS], sems, add=True)
                    if first:
                        pltpu.async_copy(wbuf, den_sh.at[didxS], sems,
                                         add=True)

                pregather(ebase, bufA)

                def body(j, _):
                    e0 = ebase + 2 * j * K1
                    pregather(e0 + K1, bufB)
                    stage(j, bufA)
                    pregather(e0 + 2 * K1, bufA)
                    stage(j, bufB)
                    return 0

                lax.fori_loop(0, nblk // 2, body, 0)
                pltpu.make_async_copy(asd_hbm.at[sidxA], sbufA, semgA).wait()
                pltpu.make_async_copy(add_hbm.at[didxA], dbufA, semgA).wait()
                pltpu.make_async_copy(
                    xlc_hbm.at[0].at[sidxA], rowsA, semgA).wait()
                pltpu.make_async_copy(srowsA, num_sh.at[didxSA], semsA).wait()
                pltpu.make_async_copy(srowsB, num_sh.at[didxSB], semsB).wait()
                if first:
                    pltpu.make_async_copy(
                        wbufA, den_sh.at[didxSA], semsA).wait()
                    pltpu.make_async_copy(
                        wbufB, den_sh.at[didxSB], semsB).wait()
                plsc.subcore_barrier()
                pltpu.sync_copy(num_sh.at[pl.ds(r0, RPT1)],
                                num_out.at[ck].at[pl.ds(r0, RPT1)])
                if first:
                    pltpu.sync_copy(den_sh.at[pl.ds(r0, RPT1)],
                                    den_out.at[pl.ds(r0, RPT1)])
                plsc.subcore_barrier()


def _l1_edge(src, dst, asd, add_, xlc, z128, zd):
    kfn = pl.kernel(
        _l1_body,
        out_type=[
            jax.ShapeDtypeStruct((4, NP, 128), jnp.float32),
            jax.ShapeDtypeStruct((NP, 16), jnp.float32),
        ],
        mesh=_sc_mesh(),
        compiler_params=pltpu.CompilerParams(use_tc_tiling_on_sc=False),
        scratch_types=(
            [pltpu.VMEM((K1,), jnp.int32),
             pltpu.VMEM((K1,), jnp.int32),
             pltpu.VMEM((K1,), jnp.int32),
             pltpu.VMEM((K1, 16), jnp.float32),
             pltpu.VMEM((K1, 16), jnp.float32),
             pltpu.VMEM((K1, 16), jnp.float32),
             pltpu.VMEM((K1, 128), jnp.float32),
             pltpu.VMEM((K1, 128), jnp.float32)] * 2
            + [pltpu.VMEM_SHARED((NP2, 128), jnp.float32),
               pltpu.VMEM_SHARED((NP2, 16), jnp.float32),
               pltpu.SemaphoreType.DMA,
               pltpu.SemaphoreType.DMA,
               pltpu.SemaphoreType.DMA,
               pltpu.SemaphoreType.DMA]),
    )
    return kfn(src, dst, asd, add_, xlc, z128, zd)


def _mid_body(num_ref, den_ref, b1_ref, w2_ref, as2_ref, ad2_ref,
              xl2_ref, asd2_ref, add2_ref):
    acc = jnp.zeros((RTM, HID), jnp.float32)
    for ck in range(4):
        nb = num_ref[ck]
        d0 = den_ref[:, 2 * ck]
        d1 = den_ref[:, 2 * ck + 1]
        div = jnp.concatenate(
            [jnp.broadcast_to(d0[:, None], (RTM, HID)),
             jnp.broadcast_to(d1[:, None], (RTM, HID))], axis=1)
        h = nb / (div + 1e-16) + b1_ref[0, 128 * ck:128 * ck + 128]
        h = jnp.maximum(h, 0.0)
        acc = acc + jnp.dot(h, w2_ref[128 * ck:128 * ck + 128, :],
                            preferred_element_type=jnp.float32)
    xl2_ref[...] = acc
    a_s = (acc * as2_ref[...]).sum(-1)
    a_d = (acc * ad2_ref[...]).sum(-1)
    asd2_ref[...] = jnp.broadcast_to(a_s[:, None], (RTM, 16))
    add2_ref[...] = jnp.broadcast_to(a_d[:, None], (RTM, 16))


def _mid(num1, den1, b1, W2, as2, ad2):
    grid = (N // RTM,)
    return pl.pallas_call(
        _mid_body,
        grid=grid,
        in_specs=[
            pl.BlockSpec((4, RTM, 128), lambda i: (0, i, 0)),
            pl.BlockSpec((RTM, 16), lambda i: (i, 0)),
            pl.BlockSpec((1, HEADS * HID), lambda i: (0, 0)),
            pl.BlockSpec((HEADS * HID, HID), lambda i: (0, 0)),
            pl.BlockSpec((1, HID), lambda i: (0, 0)),
            pl.BlockSpec((1, HID), lambda i: (0, 0)),
        ],
        out_specs=[
            pl.BlockSpec((RTM, HID), lambda i: (i, 0)),
            pl.BlockSpec((RTM, 16), lambda i: (i, 0)),
            pl.BlockSpec((RTM, 16), lambda i: (i, 0)),
        ],
        out_shape=[
            jax.ShapeDtypeStruct((NP, HID), jnp.float32),
            jax.ShapeDtypeStruct((NP, 16), jnp.float32),
            jax.ShapeDtypeStruct((NP, 16), jnp.float32),
        ],
    )(num1, den1, b1, W2, as2, ad2)


def _l2_body(src_hbm, dst_hbm, asd_hbm, add_hbm, xl2_hbm, z64_hbm, zd_hbm,
             num_out, den_out,
             sidxA, didxA, didxSA, sbufA, dbufA, wbufA, rowsA, srowsA,
             sidxB, didxB, didxSB, sbufB, dbufB, wbufB, rowsB, srowsB,
             num_sh, den_sh, semgA, semgB, semsA, semsB):
    c = lax.axis_index("c")
    s = lax.axis_index("s")
    r0 = s * ROWS_PT
    ept = EAP // (CORES * TILES)
    nblk = ept // K
    ebase = (c * TILES + s) * ept
    bufA = (sidxA, didxA, didxSA, sbufA, dbufA, wbufA, rowsA, srowsA,
            semgA, semsA)
    bufB = (sidxB, didxB, didxSB, sbufB, dbufB, wbufB, rowsB, srowsB,
            semgB, semsB)

    pltpu.sync_copy(z64_hbm.at[pl.ds(r0, ROWS_PT)],
                    num_sh.at[pl.ds(r0, ROWS_PT)])
    pltpu.sync_copy(zd_hbm.at[pl.ds(r0, ROWS_PT)],
                    den_sh.at[pl.ds(r0, ROWS_PT)])
    plsc.subcore_barrier()

    def pregather(e0, buf):
        sidx, didx, didxS, sbuf, dbuf, wbuf, rows, srows, semg, sems = buf
        pltpu.sync_copy(src_hbm.at[pl.ds(e0, K)], sidx)
        pltpu.sync_copy(dst_hbm.at[pl.ds(e0, K)], didx)
        pltpu.async_copy(asd_hbm.at[sidx], sbuf, semg)
        pltpu.async_copy(add_hbm.at[didx], dbuf, semg)
        pltpu.async_copy(xl2_hbm.at[sidx], rows, semg)

    def stage(j, buf):
        sidx, didx, didxS, sbuf, dbuf, wbuf, rows, srows, semg, sems = buf
        pltpu.make_async_copy(asd_hbm.at[sidx], sbuf, semg).wait()
        pltpu.make_async_copy(add_hbm.at[didx], dbuf, semg).wait()
        pltpu.make_async_copy(xl2_hbm.at[sidx], rows, semg).wait()

        @pl.when(j > 0)
        def _drain_prev_scatter():
            pltpu.make_async_copy(srows, num_sh.at[didxS], sems).wait()
            pltpu.make_async_copy(wbuf, den_sh.at[didxS], sems).wait()

        for t in range(K // 16):
            didxS[pl.ds(t * 16, 16)] = didx[pl.ds(t * 16, 16)]

        def edge(k, _):
            al = sbuf[k, :] + dbuf[k, :]
            al = jnp.maximum(al, 0.2 * al)
            w = jnp.exp(al)
            wbuf[k, :] = w
            w0 = w[0]
            for t in range(4):
                srows[k, pl.ds(t * 16, 16)] = rows[k, pl.ds(t * 16, 16)] * w0
            return 0

        lax.fori_loop(0, K, edge, 0)
        pltpu.async_copy(srows, num_sh.at[didxS], sems, add=True)
        pltpu.async_copy(wbuf, den_sh.at[didxS], sems, add=True)

    pregather(ebase, bufA)

    def body(j, _):
        e0 = ebase + 2 * j * K
        pregather(e0 + K, bufB)
        stage(j, bufA)
        pregather(e0 + 2 * K, bufA)
        stage(j, bufB)
        return 0

    lax.fori_loop(0, nblk // 2, body, 0)
    pltpu.make_async_copy(asd_hbm.at[sidxA], sbufA, semgA).wait()
    pltpu.make_async_copy(add_hbm.at[didxA], dbufA, semgA).wait()
    pltpu.make_async_copy(xl2_hbm.at[sidxA], rowsA, semgA).wait()
    pltpu.make_async_copy(srowsA, num_sh.at[didxSA], semsA).wait()
    pltpu.make_async_copy(wbufA, den_sh.at[didxSA], semsA).wait()
    pltpu.make_async_copy(srowsB, num_sh.at[didxSB], semsB).wait()
    pltpu.make_async_copy(wbufB, den_sh.at[didxSB], semsB).wait()
    plsc.subcore_barrier()
    pltpu.sync_copy(num_sh.at[pl.ds(r0, ROWS_PT)],
                    num_out.at[c].at[pl.ds(r0, ROWS_PT)])
    pltpu.sync_copy(den_sh.at[pl.ds(r0, ROWS_PT)],
                    den_out.at[c].at[pl.ds(r0, ROWS_PT)])


def _l2_edge(src, dst, asd2, add2, xl2, z64, zd):
    kfn = pl.kernel(
        _l2_body,
        out_type=[
            jax.ShapeDtypeStruct((2, NP, HID), jnp.float32),
            jax.ShapeDtypeStruct((2, NP, 16), jnp.float32),
        ],
        mesh=_sc_mesh(),
        compiler_params=pltpu.CompilerParams(use_tc_tiling_on_sc=False),
        scratch_types=(
            [pltpu.VMEM((K,), jnp.int32),
             pltpu.VMEM((K,), jnp.int32),
             pltpu.VMEM((K,), jnp.int32),
             pltpu.VMEM((K, 16), jnp.float32),
             pltpu.VMEM((K, 16), jnp.float32),
             pltpu.VMEM((K, 16), jnp.float32),
             pltpu.VMEM((K, HID), jnp.float32),
             pltpu.VMEM((K, HID), jnp.float32)] * 2
            + [pltpu.VMEM_SHARED((NP, HID), jnp.float32),
               pltpu.VMEM_SHARED((NP, 16), jnp.float32),
               pltpu.SemaphoreType.DMA,
               pltpu.SemaphoreType.DMA,
               pltpu.SemaphoreType.DMA,
               pltpu.SemaphoreType.DMA]),
    )
    return kfn(src, dst, asd2, add2, xl2, z64, zd)


def _post_body(num_ref, den_ref, b2_ref, batch_ref, lw1_ref, lb1_ref,
               lw2_ref, lb2_ref, lw3_ref, lb3_ref, out_ref):
    num = num_ref[0] + num_ref[1]
    den = den_ref[0][:, 0] + den_ref[1][:, 0]
    h2 = jnp.maximum(num / (den[:, None] + 1e-16) + b2_ref[...], 0.0)
    valid = lax.broadcasted_iota(jnp.int32, (NP, 1), 0) < N
    h2 = jnp.where(valid, h2, 0.0)
    onehot = (batch_ref[...] ==
              lax.broadcasted_iota(jnp.int32, (NG, NP), 0)).astype(jnp.float32)
    sums = jnp.dot(onehot, h2, preferred_element_type=jnp.float32)
    cnt = jnp.sum(onehot, axis=1)
    g = sums / jnp.maximum(cnt, 1.0)[:, None]
    g = jnp.maximum(jnp.dot(g, lw1_ref[...],
                            preferred_element_type=jnp.float32) + lb1_ref[...], 0.0)
    g = jnp.maximum(jnp.dot(g, lw2_ref[...],
                            preferred_element_type=jnp.float32) + lb2_ref[...], 0.0)
    logits = jnp.dot(g, lw3_ref[...],
                     preferred_element_type=jnp.float32) + lb3_ref[...]
    m = jnp.max(logits, axis=-1, keepdims=True)
    lse = jnp.log(jnp.sum(jnp.exp(logits - m), axis=-1, keepdims=True)) + m
    out_ref[...] = logits - lse


def _post(num2, den2, b2, batch_i, lw1, lb1, lw2, lb2, lw3, lb3):
    return pl.pallas_call(
        _post_body,
        out_shape=jax.ShapeDtypeStruct((NG, NCLS), jnp.float32),
    )(num2, den2, b2, batch_i, lw1, lb1, lw2, lb2, lw3, lb3)


def kernel(x, edge_index, batch, W1, att_src1, att_dst1, b1,
           W2, att_src2, att_dst2, b2, lw1, lb1, lw2, lb2, lw3, lb3):
    loops = jnp.arange(N, dtype=jnp.int32)
    pad = jnp.full((EAP - EA + K,), N, jnp.int32)
    src = jnp.concatenate([edge_index[0].astype(jnp.int32), loops, pad])
    dst = jnp.concatenate([edge_index[1].astype(jnp.int32), loops, pad])

    as1 = att_src1.reshape(1, HEADS * HID)
    ad1 = att_dst1.reshape(1, HEADS * HID)
    xl, asd, add_ = _pre1(x, W1, as1, ad1)
    zrows16 = jnp.zeros((NP - N, 16), jnp.float32)
    asd = jnp.concatenate([asd, zrows16])
    add_ = jnp.concatenate([add_, zrows16])
    xlc = jnp.concatenate(
        [xl, jnp.zeros((NP - N, HEADS * HID), jnp.float32)]
    ).reshape(NP, 4, 128).transpose(1, 0, 2)

    z128 = jnp.zeros((NP, 128), jnp.float32)
    z64 = jnp.zeros((NP, HID), jnp.float32)
    zd = jnp.zeros((NP, 16), jnp.float32)
    num1, den1 = _l1_edge(src, dst, asd, add_, xlc, z128, zd)

    xl2, asd2, add2 = _mid(num1, den1, b1.reshape(1, HEADS * HID), W2,
                           att_src2.reshape(1, HID), att_dst2.reshape(1, HID))
    num2, den2 = _l2_edge(src, dst, asd2, add2, xl2, z64, zd)

    return _post(num2, den2, b2.reshape(1, HID),
                 jnp.concatenate([batch.astype(jnp.int32), jnp.full((NP - N,), NG, jnp.int32)]).reshape(1, NP),
                 lw1, lb1.reshape(1, HID), lw2, lb2.reshape(1, HID),
                 lw3, lb3.reshape(1, NCLS))


# speedup vs baseline: 1.0060x; 1.0060x over previous
import functools

import jax
import jax.numpy as jnp
from jax import lax
from jax.experimental import pallas as pl
from jax.experimental.pallas import tpu as pltpu
from jax.experimental.pallas import tpu_sc as plsc

N = 10000
E = 320000
F_IN = 128
HID = 64
HEADS = 8
NCLS = 10
NG = 64

EA = E + N
K = 128
TILES = 16
CORES = 2
EAP = -(-EA // (CORES * TILES * K * 2)) * (CORES * TILES * K * 2)
NP = 10240
ROWS_PT = NP // TILES
K1 = 64
NP2 = 10112
RPT1 = NP2 // TILES
RT = 1000
RTM = 1024


def _sc_mesh():
    return plsc.VectorSubcoreMesh(
        core_axis_name="c", subcore_axis_name="s",
        num_cores=CORES, num_subcores=TILES)


def _pre1_body(x_ref, w_ref, as_ref, ad_ref, xl_ref, asd_ref, add_ref):
    xb = jnp.dot(x_ref[...], w_ref[...], preferred_element_type=jnp.float32)
    xl_ref[...] = xb
    a_s = (xb * as_ref[...]).reshape(RT, HEADS, HID).sum(-1)
    a_d = (xb * ad_ref[...]).reshape(RT, HEADS, HID).sum(-1)
    asd_ref[...] = jnp.concatenate([a_s, a_s], axis=1)
    add_ref[...] = jnp.concatenate([a_d, a_d], axis=1)


def _pre1(x, W1, as1, ad1):
    grid = (N // RT,)
    return pl.pallas_call(
        _pre1_body,
        grid=grid,
        in_specs=[
            pl.BlockSpec((RT, F_IN), lambda i: (i, 0)),
            pl.BlockSpec((F_IN, HEADS * HID), lambda i: (0, 0)),
            pl.BlockSpec((1, HEADS * HID), lambda i: (0, 0)),
            pl.BlockSpec((1, HEADS * HID), lambda i: (0, 0)),
        ],
        out_specs=[
            pl.BlockSpec((RT, HEADS * HID), lambda i: (i, 0)),
            pl.BlockSpec((RT, 16), lambda i: (i, 0)),
            pl.BlockSpec((RT, 16), lambda i: (i, 0)),
        ],
        out_shape=[
            jax.ShapeDtypeStruct((N, HEADS * HID), jnp.float32),
            jax.ShapeDtypeStruct((N, 16), jnp.float32),
            jax.ShapeDtypeStruct((N, 16), jnp.float32),
        ],
    )(x, W1, as1, ad1)


def _l1_body(src_hbm, dst_hbm, asd_hbm, add_hbm, xlc_hbm, z128_hbm, zd_hbm,
             num_out, den_out,
             sidxA, didxA, didxSA, sbufA, dbufA, wbufA, rowsA, srowsA,
             sidxB, didxB, didxSB, sbufB, dbufB, wbufB, rowsB, srowsB,
             num_sh, den_sh, semgA, semgB, semsA, semsB):
    c = lax.axis_index("c")
    s = lax.axis_index("s")
    r0 = s * RPT1
    ept = EAP // TILES
    nblk = ept // K1
    ebase = s * ept
    bufA = (sidxA, didxA, didxSA, sbufA, dbufA, wbufA, rowsA, srowsA,
            semgA, semsA)
    bufB = (sidxB, didxB, didxSB, sbufB, dbufB, wbufB, rowsB, srowsB,
            semgB, semsB)

    for cs in range(CORES):
        @pl.when(c == cs)
        def _core_branch(cs=cs):
            for ckl in range(2):
                ck = cs * 2 + ckl
                h0 = 2 * ck
                h1 = 2 * ck + 1
                first = (cs == 0 and ckl == 0)
                pltpu.sync_copy(z128_hbm.at[pl.ds(r0, RPT1)],
                                num_sh.at[pl.ds(r0, RPT1)])
                if first:
                    pltpu.sync_copy(zd_hbm.at[pl.ds(r0, RPT1)],
                                    den_sh.at[pl.ds(r0, RPT1)])
                plsc.subcore_barrier()

                def pregather(e0, buf, ck=ck):
                    sidx, didx, didxS, sbuf, dbuf, wbuf, rows, srows, \
                        semg, sems = buf
                    pltpu.sync_copy(src_hbm.at[pl.ds(e0, K1)], sidx)
                    pltpu.sync_copy(dst_hbm.at[pl.ds(e0, K1)], didx)
                    pltpu.async_copy(asd_hbm.at[sidx], sbuf, semg)
                    pltpu.async_copy(add_hbm.at[didx], dbuf, semg)
                    pltpu.async_copy(xlc_hbm.at[ck].at[sidx], rows, semg)

                def stage(j, buf, first=first, h0=h0, h1=h1):
                    sidx, didx, didxS, sbuf, dbuf, wbuf, rows, srows, \
                        semg, sems = buf
                    pltpu.make_async_copy(asd_hbm.at[sidx], sbuf, semg).wait()
                    pltpu.make_async_copy(add_hbm.at[didx], dbuf, semg).wait()
                    pltpu.make_async_copy(
                        xlc_hbm.at[0].at[sidx], rows, semg).wait()

                    @pl.when(j > 0)
                    def _drain_prev_scatter():
                        pltpu.make_async_copy(
                            srows, num_sh.at[didxS], sems).wait()
                        if first:
                            pltpu.make_async_copy(
                                wbuf, den_sh.at[didxS], sems).wait()

                    for t in range(K1 // 16):
                        didxS[pl.ds(t * 16, 16)] = didx[pl.ds(t * 16, 16)]

                    def edge8(k8, _):
                        for u in range(8):
                            k = k8 * 8 + u
                            al = sbuf[k, :] + dbuf[k, :]
                            al = jnp.maximum(al, 0.2 * al)
                            w = jnp.exp(al)
                            if first:
                                wbuf[k, :] = w
                            w0 = jnp.broadcast_to(w[h0], (16,))
                            w1 = jnp.broadcast_to(w[h1], (16,))
                            for t in range(4):
                                srows[k, pl.ds(t * 16, 16)] = (
                                    rows[k, pl.ds(t * 16, 16)] * w0)
                            for t in range(4, 8):
                                srows[k, pl.ds(t * 16, 16)] = (
                                    rows[k, pl.ds(t * 16, 16)] * w1)
                        return 0

                    lax.fori_loop(0, K1 // 8, edge8, 0)
                    pltpu.async_copy(srows, num_sh.at[didxS], sems, add=True)
                    if first:
                        pltpu.async_copy(wbuf, den_sh.at[didxS], sems,
                                         add=True)

                pregather(ebase, bufA)

                def body(j, _):
                    e0 = ebase + 2 * j * K1
                    pregather(e0 + K1, bufB)
                    stage(j, bufA)
                    pregather(e0 + 2 * K1, bufA)
                    stage(j, bufB)
                    return 0

                lax.fori_loop(0, nblk // 2, body, 0)
                pltpu.make_async_copy(asd_hbm.at[sidxA], sbufA, semgA).wait()
                pltpu.make_async_copy(add_hbm.at[didxA], dbufA, semgA).wait()
                pltpu.make_async_copy(
                    xlc_hbm.at[0].at[sidxA], rowsA, semgA).wait()
                pltpu.make_async_copy(srowsA, num_sh.at[didxSA], semsA).wait()
                pltpu.make_async_copy(srowsB, num_sh.at[didxSB], semsB).wait()
                if first:
                    pltpu.make_async_copy(
                        wbufA, den_sh.at[didxSA], semsA).wait()
                    pltpu.make_async_copy(
                        wbufB, den_sh.at[didxSB], semsB).wait()
                plsc.subcore_barrier()
                pltpu.sync_copy(num_sh.at[pl.ds(r0, RPT1)],
                                num_out.at[ck].at[pl.ds(r0, RPT1)])
                if first:
                    pltpu.sync_copy(den_sh.at[pl.ds(r0, RPT1)],
                                    den_out.at[pl.ds(r0, RPT1)])
                plsc.subcore_barrier()


def _l1_edge(src, dst, asd, add_, xlc, z128, zd):
    kfn = pl.kernel(
        _l1_body,
        out_type=[
            jax.ShapeDtypeStruct((4, NP, 128), jnp.float32),
            jax.ShapeDtypeStruct((NP, 16), jnp.float32),
        ],
        mesh=_sc_mesh(),
        compiler_params=pltpu.CompilerParams(use_tc_tiling_on_sc=False),
        scratch_types=(
            [pltpu.VMEM((K1,), jnp.int32),
             pltpu.VMEM((K1,), jnp.int32),
             pltpu.VMEM((K1,), jnp.int32),
             pltpu.VMEM((K1, 16), jnp.float32),
             pltpu.VMEM((K1, 16), jnp.float32),
             pltpu.VMEM((K1, 16), jnp.float32),
             pltpu.VMEM((K1, 128), jnp.float32),
             pltpu.VMEM((K1, 128), jnp.float32)] * 2
            + [pltpu.VMEM_SHARED((NP2, 128), jnp.float32),
               pltpu.VMEM_SHARED((NP2, 16), jnp.float32),
               pltpu.SemaphoreType.DMA,
               pltpu.SemaphoreType.DMA,
               pltpu.SemaphoreType.DMA,
               pltpu.SemaphoreType.DMA]),
    )
    return kfn(src, dst, asd, add_, xlc, z128, zd)


def _mid_body(num_ref, den_ref, b1_ref, w2_ref, as2_ref, ad2_ref,
              xl2_ref, asd2_ref, add2_ref):
    acc = jnp.zeros((RTM, HID), jnp.float32)
    for ck in range(4):
        nb = num_ref[ck]
        d0 = den_ref[:, 2 * ck]
        d1 = den_ref[:, 2 * ck + 1]
        div = jnp.concatenate(
            [jnp.broadcast_to(d0[:, None], (RTM, HID)),
             jnp.broadcast_to(d1[:, None], (RTM, HID))], axis=1)
        h = nb / (div + 1e-16) + b1_ref[0, 128 * ck:128 * ck + 128]
        h = jnp.maximum(h, 0.0)
        acc = acc + jnp.dot(h, w2_ref[128 * ck:128 * ck + 128, :],
                            preferred_element_type=jnp.float32)
    xl2_ref[...] = acc
    a_s = (acc * as2_ref[...]).sum(-1)
    a_d = (acc * ad2_ref[...]).sum(-1)
    asd2_ref[...] = jnp.broadcast_to(a_s[:, None], (RTM, 16))
    add2_ref[...] = jnp.broadcast_to(a_d[:, None], (RTM, 16))


def _mid(num1, den1, b1, W2, as2, ad2):
    grid = (N // RTM,)
    return pl.pallas_call(
        _mid_body,
        grid=grid,
        in_specs=[
            pl.BlockSpec((4, RTM, 128), lambda i: (0, i, 0)),
            pl.BlockSpec((RTM, 16), lambda i: (i, 0)),
            pl.BlockSpec((1, HEADS * HID), lambda i: (0, 0)),
            pl.BlockSpec((HEADS * HID, HID), lambda i: (0, 0)),
            pl.BlockSpec((1, HID), lambda i: (0, 0)),
            pl.BlockSpec((1, HID), lambda i: (0, 0)),
        ],
        out_specs=[
            pl.BlockSpec((RTM, HID), lambda i: (i, 0)),
            pl.BlockSpec((RTM, 16), lambda i: (i, 0)),
            pl.BlockSpec((RTM, 16), lambda i: (i, 0)),
        ],
        out_shape=[
            jax.ShapeDtypeStruct((NP, HID), jnp.float32),
            jax.ShapeDtypeStruct((NP, 16), jnp.float32),
            jax.ShapeDtypeStruct((NP, 16), jnp.float32),
        ],
    )(num1, den1, b1, W2, as2, ad2)


def _l2_body(src_hbm, dst_hbm, asd_hbm, add_hbm, xl2_hbm, z64_hbm, zd_hbm,
             num_out, den_out,
             sidxA, didxA, didxSA, sbufA, dbufA, wbufA, rowsA, srowsA,
             sidxB, didxB, didxSB, sbufB, dbufB, wbufB, rowsB, srowsB,
             num_sh, den_sh, semgA, semgB, semsA, semsB):
    c = lax.axis_index("c")
    s = lax.axis_index("s")
    r0 = s * ROWS_PT
    ept = EAP // (CORES * TILES)
    nblk = ept // K
    ebase = (c * TILES + s) * ept
    bufA = (sidxA, didxA, didxSA, sbufA, dbufA, wbufA, rowsA, srowsA,
            semgA, semsA)
    bufB = (sidxB, didxB, didxSB, sbufB, dbufB, wbufB, rowsB, srowsB,
            semgB, semsB)

    pltpu.sync_copy(z64_hbm.at[pl.ds(r0, ROWS_PT)],
                    num_sh.at[pl.ds(r0, ROWS_PT)])
    pltpu.sync_copy(zd_hbm.at[pl.ds(r0, ROWS_PT)],
                    den_sh.at[pl.ds(r0, ROWS_PT)])
    plsc.subcore_barrier()

    def pregather(e0, buf):
        sidx, didx, didxS, sbuf, dbuf, wbuf, rows, srows, semg, sems = buf
        pltpu.sync_copy(src_hbm.at[pl.ds(e0, K)], sidx)
        pltpu.sync_copy(dst_hbm.at[pl.ds(e0, K)], didx)
        pltpu.async_copy(asd_hbm.at[sidx], sbuf, semg)
        pltpu.async_copy(add_hbm.at[didx], dbuf, semg)
        pltpu.async_copy(xl2_hbm.at[sidx], rows, semg)

    def stage(j, buf):
        sidx, didx, didxS, sbuf, dbuf, wbuf, rows, srows, semg, sems = buf
        pltpu.make_async_copy(asd_hbm.at[sidx], sbuf, semg).wait()
        pltpu.make_async_copy(add_hbm.at[didx], dbuf, semg).wait()
        pltpu.make_async_copy(xl2_hbm.at[sidx], rows, semg).wait()

        @pl.when(j > 0)
        def _drain_prev_scatter():
            pltpu.make_async_copy(srows, num_sh.at[didxS], sems).wait()
            pltpu.make_async_copy(wbuf, den_sh.at[didxS], sems).wait()

        for t in range(K // 16):
            didxS[pl.ds(t * 16, 16)] = didx[pl.ds(t * 16, 16)]

        def edge8(k8, _):
            for u in range(8):
                k = k8 * 8 + u
                al = sbuf[k, :] + dbuf[k, :]
                al = jnp.maximum(al, 0.2 * al)
                w = jnp.exp(al)
                wbuf[k, :] = w
                w0 = jnp.broadcast_to(w[0], (16,))
                for t in range(4):
                    srows[k, pl.ds(t * 16, 16)] = (
                        rows[k, pl.ds(t * 16, 16)] * w0)
            return 0

        lax.fori_loop(0, K // 8, edge8, 0)
        pltpu.async_copy(srows, num_sh.at[didxS], sems, add=True)
        pltpu.async_copy(wbuf, den_sh.at[didxS], sems, add=True)

    pregather(ebase, bufA)

    def body(j, _):
        e0 = ebase + 2 * j * K
        pregather(e0 + K, bufB)
        stage(j, bufA)
        pregather(e0 + 2 * K, bufA)
        stage(j, bufB)
        return 0

    lax.fori_loop(0, nblk // 2, body, 0)
    pltpu.make_async_copy(asd_hbm.at[sidxA], sbufA, semgA).wait()
    pltpu.make_async_copy(add_hbm.at[didxA], dbufA, semgA).wait()
    pltpu.make_async_copy(xl2_hbm.at[sidxA], rowsA, semgA).wait()
    pltpu.make_async_copy(srowsA, num_sh.at[didxSA], semsA).wait()
    pltpu.make_async_copy(wbufA, den_sh.at[didxSA], semsA).wait()
    pltpu.make_async_copy(srowsB, num_sh.at[didxSB], semsB).wait()
    pltpu.make_async_copy(wbufB, den_sh.at[didxSB], semsB).wait()
    plsc.subcore_barrier()
    pltpu.sync_copy(num_sh.at[pl.ds(r0, ROWS_PT)],
                    num_out.at[c].at[pl.ds(r0, ROWS_PT)])
    pltpu.sync_copy(den_sh.at[pl.ds(r0, ROWS_PT)],
                    den_out.at[c].at[pl.ds(r0, ROWS_PT)])


def _l2_edge(src, dst, asd2, add2, xl2, z64, zd):
    kfn = pl.kernel(
        _l2_body,
        out_type=[
            jax.ShapeDtypeStruct((2, NP, HID), jnp.float32),
            jax.ShapeDtypeStruct((2, NP, 16), jnp.float32),
        ],
        mesh=_sc_mesh(),
        compiler_params=pltpu.CompilerParams(use_tc_tiling_on_sc=False),
        scratch_types=(
            [pltpu.VMEM((K,), jnp.int32),
             pltpu.VMEM((K,), jnp.int32),
             pltpu.VMEM((K,), jnp.int32),
             pltpu.VMEM((K, 16), jnp.float32),
             pltpu.VMEM((K, 16), jnp.float32),
             pltpu.VMEM((K, 16), jnp.float32),
             pltpu.VMEM((K, HID), jnp.float32),
             pltpu.VMEM((K, HID), jnp.float32)] * 2
            + [pltpu.VMEM_SHARED((NP, HID), jnp.float32),
               pltpu.VMEM_SHARED((NP, 16), jnp.float32),
               pltpu.SemaphoreType.DMA,
               pltpu.SemaphoreType.DMA,
               pltpu.SemaphoreType.DMA,
               pltpu.SemaphoreType.DMA]),
    )
    return kfn(src, dst, asd2, add2, xl2, z64, zd)


def _post_body(num_ref, den_ref, b2_ref, batch_ref, lw1_ref, lb1_ref,
               lw2_ref, lb2_ref, lw3_ref, lb3_ref, out_ref):
    num = num_ref[0] + num_ref[1]
    den = den_ref[0][:, 0] + den_ref[1][:, 0]
    h2 = jnp.maximum(num / (den[:, None] + 1e-16) + b2_ref[...], 0.0)
    valid = lax.broadcasted_iota(jnp.int32, (NP, 1), 0) < N
    h2 = jnp.where(valid, h2, 0.0)
    onehot = (batch_ref[...] ==
              lax.broadcasted_iota(jnp.int32, (NG, NP), 0)).astype(jnp.float32)
    sums = jnp.dot(onehot, h2, preferred_element_type=jnp.float32)
    cnt = jnp.sum(onehot, axis=1)
    g = sums / jnp.maximum(cnt, 1.0)[:, None]
    g = jnp.maximum(jnp.dot(g, lw1_ref[...],
                            preferred_element_type=jnp.float32) + lb1_ref[...], 0.0)
    g = jnp.maximum(jnp.dot(g, lw2_ref[...],
                            preferred_element_type=jnp.float32) + lb2_ref[...], 0.0)
    logits = jnp.dot(g, lw3_ref[...],
                     preferred_element_type=jnp.float32) + lb3_ref[...]
    m = jnp.max(logits, axis=-1, keepdims=True)
    lse = jnp.log(jnp.sum(jnp.exp(logits - m), axis=-1, keepdims=True)) + m
    out_ref[...] = logits - lse


def _post(num2, den2, b2, batch_i, lw1, lb1, lw2, lb2, lw3, lb3):
    return pl.pallas_call(
        _post_body,
        out_shape=jax.ShapeDtypeStruct((NG, NCLS), jnp.float32),
    )(num2, den2, b2, batch_i, lw1, lb1, lw2, lb2, lw3, lb3)


def kernel(x, edge_index, batch, W1, att_src1, att_dst1, b1,
           W2, att_src2, att_dst2, b2, lw1, lb1, lw2, lb2, lw3, lb3):
    loops = jnp.arange(N, dtype=jnp.int32)
    pad = jnp.full((EAP - EA + K,), N, jnp.int32)
    src = jnp.concatenate([edge_index[0].astype(jnp.int32), loops, pad])
    dst = jnp.concatenate([edge_index[1].astype(jnp.int32), loops, pad])

    as1 = att_src1.reshape(1, HEADS * HID)
    ad1 = att_dst1.reshape(1, HEADS * HID)
    xl, asd, add_ = _pre1(x, W1, as1, ad1)
    zrows16 = jnp.zeros((NP - N, 16), jnp.float32)
    asd = jnp.concatenate([asd, zrows16])
    add_ = jnp.concatenate([add_, zrows16])
    xlc = jnp.concatenate(
        [xl, jnp.zeros((NP - N, HEADS * HID), jnp.float32)]
    ).reshape(NP, 4, 128).transpose(1, 0, 2)

    z128 = jnp.zeros((NP, 128), jnp.float32)
    z64 = jnp.zeros((NP, HID), jnp.float32)
    zd = jnp.zeros((NP, 16), jnp.float32)
    num1, den1 = _l1_edge(src, dst, asd, add_, xlc, z128, zd)

    xl2, asd2, add2 = _mid(num1, den1, b1.reshape(1, HEADS * HID), W2,
                           att_src2.reshape(1, HID), att_dst2.reshape(1, HID))
    num2, den2 = _l2_edge(src, dst, asd2, add2, xl2, z64, zd)

    return _post(num2, den2, b2.reshape(1, HID),
                 jnp.concatenate([batch.astype(jnp.int32), jnp.full((NP - N,), NG, jnp.int32)]).reshape(1, NP),
                 lw1, lb1.reshape(1, HID), lw2, lb2.reshape(1, HID),
                 lw3, lb3.reshape(1, NCLS))


# speedup vs baseline: 1.1828x; 1.1757x over previous
import functools

import jax
import jax.numpy as jnp
from jax import lax
from jax.experimental import pallas as pl
from jax.experimental.pallas import tpu as pltpu
from jax.experimental.pallas import tpu_sc as plsc

N = 10000
E = 320000
F_IN = 128
HID = 64
HEADS = 8
NCLS = 10
NG = 64

EA = E + N
K = 128
TILES = 16
CORES = 2
EAP = -(-EA // (CORES * TILES * K * 2)) * (CORES * TILES * K * 2)
NP = 10240
ROWS_PT = NP // TILES
K1 = 64
NP2 = 10112
RPT1 = NP2 // TILES
RT = 1000
RTM = 1024


def _sc_mesh():
    return plsc.VectorSubcoreMesh(
        core_axis_name="c", subcore_axis_name="s",
        num_cores=CORES, num_subcores=TILES)


def _pre1_body(x_ref, w_ref, as_ref, ad_ref, xl_ref, asd_ref, add_ref):
    xb = jnp.dot(x_ref[...], w_ref[...], preferred_element_type=jnp.float32)
    xl_ref[...] = xb
    a_s = (xb * as_ref[...]).reshape(RT, HEADS, HID).sum(-1)
    a_d = (xb * ad_ref[...]).reshape(RT, HEADS, HID).sum(-1)
    asd_ref[...] = jnp.concatenate([a_s, a_s], axis=1)
    add_ref[...] = jnp.concatenate([a_d, a_d], axis=1)


def _pre1(x, W1, as1, ad1):
    grid = (N // RT,)
    return pl.pallas_call(
        _pre1_body,
        grid=grid,
        in_specs=[
            pl.BlockSpec((RT, F_IN), lambda i: (i, 0)),
            pl.BlockSpec((F_IN, HEADS * HID), lambda i: (0, 0)),
            pl.BlockSpec((1, HEADS * HID), lambda i: (0, 0)),
            pl.BlockSpec((1, HEADS * HID), lambda i: (0, 0)),
        ],
        out_specs=[
            pl.BlockSpec((RT, HEADS * HID), lambda i: (i, 0)),
            pl.BlockSpec((RT, 16), lambda i: (i, 0)),
            pl.BlockSpec((RT, 16), lambda i: (i, 0)),
        ],
        out_shape=[
            jax.ShapeDtypeStruct((N, HEADS * HID), jnp.float32),
            jax.ShapeDtypeStruct((N, 16), jnp.float32),
            jax.ShapeDtypeStruct((N, 16), jnp.float32),
        ],
    )(x, W1, as1, ad1)


def _l1_body(src_hbm, dst_hbm, asd_hbm, add_hbm, xlc_hbm, z128_hbm, zd_hbm,
             num_out, den_out,
             sidxA, didxA, didxSA, sbufA, dbufA, wbufA, rowsA, srowsA,
             sidxB, didxB, didxSB, sbufB, dbufB, wbufB, rowsB, srowsB,
             num_sh, den_sh, semgA, semgB, semsA, semsB):
    c = lax.axis_index("c")
    s = lax.axis_index("s")
    r0 = s * RPT1
    ept = EAP // TILES
    nblk = ept // K1
    ebase = s * ept
    bufA = (sidxA, didxA, didxSA, sbufA, dbufA, wbufA, rowsA, srowsA,
            semgA, semsA)
    bufB = (sidxB, didxB, didxSB, sbufB, dbufB, wbufB, rowsB, srowsB,
            semgB, semsB)

    for cs in range(CORES):
        @pl.when(c == cs)
        def _core_branch(cs=cs):
            for ckl in range(2):
                ck = cs * 2 + ckl
                h0 = 2 * ck
                h1 = 2 * ck + 1
                first = (cs == 0 and ckl == 0)
                pltpu.sync_copy(z128_hbm.at[pl.ds(r0, RPT1)],
                                num_sh.at[pl.ds(r0, RPT1)])
                if first:
                    pltpu.sync_copy(zd_hbm.at[pl.ds(r0, RPT1)],
                                    den_sh.at[pl.ds(r0, RPT1)])
                plsc.subcore_barrier()

                def pregather(e0, buf, ck=ck):
                    sidx, didx, didxS, sbuf, dbuf, wbuf, rows, srows, \
                        semg, sems = buf
                    pltpu.sync_copy(src_hbm.at[pl.ds(e0, K1)], sidx)
                    pltpu.sync_copy(dst_hbm.at[pl.ds(e0, K1)], didx)
                    pltpu.async_copy(asd_hbm.at[sidx], sbuf, semg)
                    pltpu.async_copy(add_hbm.at[didx], dbuf, semg)
                    pltpu.async_copy(xlc_hbm.at[ck].at[sidx], rows, semg)

                def stage(j, buf, first=first, h0=h0, h1=h1):
                    sidx, didx, didxS, sbuf, dbuf, wbuf, rows, srows, \
                        semg, sems = buf
                    pltpu.make_async_copy(asd_hbm.at[sidx], sbuf, semg).wait()
                    pltpu.make_async_copy(add_hbm.at[didx], dbuf, semg).wait()
                    pltpu.make_async_copy(
                        xlc_hbm.at[0].at[sidx], rows, semg).wait()

                    @pl.when(j > 0)
                    def _drain_prev_scatter():
                        pltpu.make_async_copy(
                            srows, num_sh.at[didxS], sems).wait()
                        if first:
                            pltpu.make_async_copy(
                                wbuf, den_sh.at[didxS], sems).wait()

                    for t in range(K1 // 16):
                        didxS[pl.ds(t * 16, 16)] = didx[pl.ds(t * 16, 16)]

                    def edge8(k8, _):
                        for u in range(8):
                            k = k8 * 8 + u
                            al = sbuf[k, :] + dbuf[k, :]
                            al = jnp.maximum(al, 0.2 * al)
                            w = jnp.exp(al)
                            if first:
                                wbuf[k, :] = w
                            w0 = jnp.broadcast_to(w[h0], (16,))
                            w1 = jnp.broadcast_to(w[h1], (16,))
                            for t in range(4):
                                wv = w0 if t < 2 else w1
                                rb = rows[k, pl.ds(t * 32, 32)]
                                lo, hi = plsc.unpack(
                                    rb, format=plsc.PackFormat.INTERLEAVED)
                                srows[k, pl.ds(t * 32, 16)] = lo * wv
                                srows[k, pl.ds(t * 32 + 16, 16)] = hi * wv
                        return 0

                    lax.fori_loop(0, K1 // 8, edge8, 0)
                    pltpu.async_copy(srows, num_sh.at[didxS], sems, add=True)
                    if first:
                        pltpu.async_copy(wbuf, den_sh.at[didxS], sems,
                                         add=True)

                pregather(ebase, bufA)

                def body(j, _):
                    e0 = ebase + 2 * j * K1
                    pregather(e0 + K1, bufB)
                    stage(j, bufA)
                    pregather(e0 + 2 * K1, bufA)
                    stage(j, bufB)
                    return 0

                lax.fori_loop(0, nblk // 2, body, 0)
                pltpu.make_async_copy(asd_hbm.at[sidxA], sbufA, semgA).wait()
                pltpu.make_async_copy(add_hbm.at[didxA], dbufA, semgA).wait()
                pltpu.make_async_copy(
                    xlc_hbm.at[0].at[sidxA], rowsA, semgA).wait()
                pltpu.make_async_copy(srowsA, num_sh.at[didxSA], semsA).wait()
                pltpu.make_async_copy(srowsB, num_sh.at[didxSB], semsB).wait()
                if first:
                    pltpu.make_async_copy(
                        wbufA, den_sh.at[didxSA], semsA).wait()
                    pltpu.make_async_copy(
                        wbufB, den_sh.at[didxSB], semsB).wait()
                plsc.subcore_barrier()
                pltpu.sync_copy(num_sh.at[pl.ds(r0, RPT1)],
                                num_out.at[ck].at[pl.ds(r0, RPT1)])
                if first:
                    pltpu.sync_copy(den_sh.at[pl.ds(r0, RPT1)],
                                    den_out.at[pl.ds(r0, RPT1)])
                plsc.subcore_barrier()


def _l1_edge(src, dst, asd, add_, xlc, z128, zd):
    kfn = pl.kernel(
        _l1_body,
        out_type=[
            jax.ShapeDtypeStruct((4, NP, 128), jnp.float32),
            jax.ShapeDtypeStruct((NP, 16), jnp.float32),
        ],
        mesh=_sc_mesh(),
        compiler_params=pltpu.CompilerParams(
            use_tc_tiling_on_sc=False, needs_layout_passes=False),
        scratch_types=(
            [pltpu.VMEM((K1,), jnp.int32),
             pltpu.VMEM((K1,), jnp.int32),
             pltpu.VMEM((K1,), jnp.int32),
             pltpu.VMEM((K1, 16), jnp.float32),
             pltpu.VMEM((K1, 16), jnp.float32),
             pltpu.VMEM((K1, 16), jnp.float32),
             pltpu.VMEM((K1, 128), jnp.bfloat16),
             pltpu.VMEM((K1, 128), jnp.float32)] * 2
            + [pltpu.VMEM_SHARED((NP2, 128), jnp.float32),
               pltpu.VMEM_SHARED((NP2, 16), jnp.float32),
               pltpu.SemaphoreType.DMA,
               pltpu.SemaphoreType.DMA,
               pltpu.SemaphoreType.DMA,
               pltpu.SemaphoreType.DMA]),
    )
    return kfn(src, dst, asd, add_, xlc, z128, zd)


def _mid_body(num_ref, den_ref, b1_ref, w2_ref, as2_ref, ad2_ref,
              xl2_ref, asd2_ref, add2_ref):
    acc = jnp.zeros((RTM, HID), jnp.float32)
    for ck in range(4):
        nb = num_ref[ck]
        d0 = den_ref[:, 2 * ck]
        d1 = den_ref[:, 2 * ck + 1]
        div = jnp.concatenate(
            [jnp.broadcast_to(d0[:, None], (RTM, HID)),
             jnp.broadcast_to(d1[:, None], (RTM, HID))], axis=1)
        h = nb / (div + 1e-16) + b1_ref[0, 128 * ck:128 * ck + 128]
        h = jnp.maximum(h, 0.0)
        acc = acc + jnp.dot(h, w2_ref[128 * ck:128 * ck + 128, :],
                            preferred_element_type=jnp.float32)
    xl2_ref[...] = acc
    a_s = (acc * as2_ref[...]).sum(-1)
    a_d = (acc * ad2_ref[...]).sum(-1)
    asd2_ref[...] = jnp.broadcast_to(a_s[:, None], (RTM, 16))
    add2_ref[...] = jnp.broadcast_to(a_d[:, None], (RTM, 16))


def _mid(num1, den1, b1, W2, as2, ad2):
    grid = (N // RTM,)
    return pl.pallas_call(
        _mid_body,
        grid=grid,
        in_specs=[
            pl.BlockSpec((4, RTM, 128), lambda i: (0, i, 0)),
            pl.BlockSpec((RTM, 16), lambda i: (i, 0)),
            pl.BlockSpec((1, HEADS * HID), lambda i: (0, 0)),
            pl.BlockSpec((HEADS * HID, HID), lambda i: (0, 0)),
            pl.BlockSpec((1, HID), lambda i: (0, 0)),
            pl.BlockSpec((1, HID), lambda i: (0, 0)),
        ],
        out_specs=[
            pl.BlockSpec((RTM, HID), lambda i: (i, 0)),
            pl.BlockSpec((RTM, 16), lambda i: (i, 0)),
            pl.BlockSpec((RTM, 16), lambda i: (i, 0)),
        ],
        out_shape=[
            jax.ShapeDtypeStruct((NP, HID), jnp.float32),
            jax.ShapeDtypeStruct((NP, 16), jnp.float32),
            jax.ShapeDtypeStruct((NP, 16), jnp.float32),
        ],
    )(num1, den1, b1, W2, as2, ad2)


def _l2_body(src_hbm, dst_hbm, asd_hbm, add_hbm, xl2_hbm, z64_hbm, zd_hbm,
             num_out, den_out,
             sidxA, didxA, didxSA, sbufA, dbufA, wbufA, rowsA, srowsA,
             sidxB, didxB, didxSB, sbufB, dbufB, wbufB, rowsB, srowsB,
             num_sh, den_sh, semgA, semgB, semsA, semsB):
    c = lax.axis_index("c")
    s = lax.axis_index("s")
    r0 = s * ROWS_PT
    ept = EAP // (CORES * TILES)
    nblk = ept // K
    ebase = (c * TILES + s) * ept
    bufA = (sidxA, didxA, didxSA, sbufA, dbufA, wbufA, rowsA, srowsA,
            semgA, semsA)
    bufB = (sidxB, didxB, didxSB, sbufB, dbufB, wbufB, rowsB, srowsB,
            semgB, semsB)

    pltpu.sync_copy(z64_hbm.at[pl.ds(r0, ROWS_PT)],
                    num_sh.at[pl.ds(r0, ROWS_PT)])
    pltpu.sync_copy(zd_hbm.at[pl.ds(r0, ROWS_PT)],
                    den_sh.at[pl.ds(r0, ROWS_PT)])
    plsc.subcore_barrier()

    def pregather(e0, buf):
        sidx, didx, didxS, sbuf, dbuf, wbuf, rows, srows, semg, sems = buf
        pltpu.sync_copy(src_hbm.at[pl.ds(e0, K)], sidx)
        pltpu.sync_copy(dst_hbm.at[pl.ds(e0, K)], didx)
        pltpu.async_copy(asd_hbm.at[sidx], sbuf, semg)
        pltpu.async_copy(add_hbm.at[didx], dbuf, semg)
        pltpu.async_copy(xl2_hbm.at[sidx], rows, semg)

    def stage(j, buf):
        sidx, didx, didxS, sbuf, dbuf, wbuf, rows, srows, semg, sems = buf
        pltpu.make_async_copy(asd_hbm.at[sidx], sbuf, semg).wait()
        pltpu.make_async_copy(add_hbm.at[didx], dbuf, semg).wait()
        pltpu.make_async_copy(xl2_hbm.at[sidx], rows, semg).wait()

        @pl.when(j > 0)
        def _drain_prev_scatter():
            pltpu.make_async_copy(srows, num_sh.at[didxS], sems).wait()
            pltpu.make_async_copy(wbuf, den_sh.at[didxS], sems).wait()

        for t in range(K // 16):
            didxS[pl.ds(t * 16, 16)] = didx[pl.ds(t * 16, 16)]

        def edge8(k8, _):
            for u in range(8):
                k = k8 * 8 + u
                al = sbuf[k, :] + dbuf[k, :]
                al = jnp.maximum(al, 0.2 * al)
                w = jnp.exp(al)
                wbuf[k, :] = w
                w0 = jnp.broadcast_to(w[0], (16,))
                for t in range(4):
                    srows[k, pl.ds(t * 16, 16)] = (
                        rows[k, pl.ds(t * 16, 16)] * w0)
            return 0

        lax.fori_loop(0, K // 8, edge8, 0)
        pltpu.async_copy(srows, num_sh.at[didxS], sems, add=True)
        pltpu.async_copy(wbuf, den_sh.at[didxS], sems, add=True)

    pregather(ebase, bufA)

    def body(j, _):
        e0 = ebase + 2 * j * K
        pregather(e0 + K, bufB)
        stage(j, bufA)
        pregather(e0 + 2 * K, bufA)
        stage(j, bufB)
        return 0

    lax.fori_loop(0, nblk // 2, body, 0)
    pltpu.make_async_copy(asd_hbm.at[sidxA], sbufA, semgA).wait()
    pltpu.make_async_copy(add_hbm.at[didxA], dbufA, semgA).wait()
    pltpu.make_async_copy(xl2_hbm.at[sidxA], rowsA, semgA).wait()
    pltpu.make_async_copy(srowsA, num_sh.at[didxSA], semsA).wait()
    pltpu.make_async_copy(wbufA, den_sh.at[didxSA], semsA).wait()
    pltpu.make_async_copy(srowsB, num_sh.at[didxSB], semsB).wait()
    pltpu.make_async_copy(wbufB, den_sh.at[didxSB], semsB).wait()
    plsc.subcore_barrier()
    pltpu.sync_copy(num_sh.at[pl.ds(r0, ROWS_PT)],
                    num_out.at[c].at[pl.ds(r0, ROWS_PT)])
    pltpu.sync_copy(den_sh.at[pl.ds(r0, ROWS_PT)],
                    den_out.at[c].at[pl.ds(r0, ROWS_PT)])


def _l2_edge(src, dst, asd2, add2, xl2, z64, zd):
    kfn = pl.kernel(
        _l2_body,
        out_type=[
            jax.ShapeDtypeStruct((2, NP, HID), jnp.float32),
            jax.ShapeDtypeStruct((2, NP, 16), jnp.float32),
        ],
        mesh=_sc_mesh(),
        compiler_params=pltpu.CompilerParams(use_tc_tiling_on_sc=False),
        scratch_types=(
            [pltpu.VMEM((K,), jnp.int32),
             pltpu.VMEM((K,), jnp.int32),
             pltpu.VMEM((K,), jnp.int32),
             pltpu.VMEM((K, 16), jnp.float32),
             pltpu.VMEM((K, 16), jnp.float32),
             pltpu.VMEM((K, 16), jnp.float32),
             pltpu.VMEM((K, HID), jnp.float32),
             pltpu.VMEM((K, HID), jnp.float32)] * 2
            + [pltpu.VMEM_SHARED((NP, HID), jnp.float32),
               pltpu.VMEM_SHARED((NP, 16), jnp.float32),
               pltpu.SemaphoreType.DMA,
               pltpu.SemaphoreType.DMA,
               pltpu.SemaphoreType.DMA,
               pltpu.SemaphoreType.DMA]),
    )
    return kfn(src, dst, asd2, add2, xl2, z64, zd)


def _post_body(num_ref, den_ref, b2_ref, batch_ref, lw1_ref, lb1_ref,
               lw2_ref, lb2_ref, lw3_ref, lb3_ref, out_ref):
    num = num_ref[0] + num_ref[1]
    den = den_ref[0][:, 0] + den_ref[1][:, 0]
    h2 = jnp.maximum(num / (den[:, None] + 1e-16) + b2_ref[...], 0.0)
    valid = lax.broadcasted_iota(jnp.int32, (NP, 1), 0) < N
    h2 = jnp.where(valid, h2, 0.0)
    onehot = (batch_ref[...] ==
              lax.broadcasted_iota(jnp.int32, (NG, NP), 0)).astype(jnp.float32)
    sums = jnp.dot(onehot, h2, preferred_element_type=jnp.float32)
    cnt = jnp.sum(onehot, axis=1)
    g = sums / jnp.maximum(cnt, 1.0)[:, None]
    g = jnp.maximum(jnp.dot(g, lw1_ref[...],
                            preferred_element_type=jnp.float32) + lb1_ref[...], 0.0)
    g = jnp.maximum(jnp.dot(g, lw2_ref[...],
                            preferred_element_type=jnp.float32) + lb2_ref[...], 0.0)
    logits = jnp.dot(g, lw3_ref[...],
                     preferred_element_type=jnp.float32) + lb3_ref[...]
    m = jnp.max(logits, axis=-1, keepdims=True)
    lse = jnp.log(jnp.sum(jnp.exp(logits - m), axis=-1, keepdims=True)) + m
    out_ref[...] = logits - lse


def _post(num2, den2, b2, batch_i, lw1, lb1, lw2, lb2, lw3, lb3):
    return pl.pallas_call(
        _post_body,
        out_shape=jax.ShapeDtypeStruct((NG, NCLS), jnp.float32),
    )(num2, den2, b2, batch_i, lw1, lb1, lw2, lb2, lw3, lb3)


def kernel(x, edge_index, batch, W1, att_src1, att_dst1, b1,
           W2, att_src2, att_dst2, b2, lw1, lb1, lw2, lb2, lw3, lb3):
    loops = jnp.arange(N, dtype=jnp.int32)
    pad = jnp.full((EAP - EA + K,), N, jnp.int32)
    src = jnp.concatenate([edge_index[0].astype(jnp.int32), loops, pad])
    dst = jnp.concatenate([edge_index[1].astype(jnp.int32), loops, pad])

    as1 = att_src1.reshape(1, HEADS * HID)
    ad1 = att_dst1.reshape(1, HEADS * HID)
    xl, asd, add_ = _pre1(x, W1, as1, ad1)
    zrows16 = jnp.zeros((NP - N, 16), jnp.float32)
    asd = jnp.concatenate([asd, zrows16])
    add_ = jnp.concatenate([add_, zrows16])
    xlp = jnp.concatenate(
        [xl, jnp.zeros((NP - N, HEADS * HID), jnp.float32)])
    xli = xlp.reshape(NP, 16, 2, 16).transpose(0, 1, 3, 2).reshape(NP, 512)
    xlc = xli.astype(jnp.bfloat16).reshape(NP, 4, 128).transpose(1, 0, 2)

    z128 = jnp.zeros((NP, 128), jnp.float32)
    z64 = jnp.zeros((NP, HID), jnp.float32)
    zd = jnp.zeros((NP, 16), jnp.float32)
    num1, den1 = _l1_edge(src, dst, asd, add_, xlc, z128, zd)

    xl2, asd2, add2 = _mid(num1, den1, b1.reshape(1, HEADS * HID), W2,
                           att_src2.reshape(1, HID), att_dst2.reshape(1, HID))
    num2, den2 = _l2_edge(src, dst, asd2, add2, xl2, z64, zd)

    return _post(num2, den2, b2.reshape(1, HID),
                 jnp.concatenate([batch.astype(jnp.int32), jnp.full((NP - N,), NG, jnp.int32)]).reshape(1, NP),
                 lw1, lb1.reshape(1, HID), lw2, lb2.reshape(1, HID),
                 lw3, lb3.reshape(1, NCLS))


# speedup vs baseline: 1.2094x; 1.0225x over previous
import functools

import jax
import jax.numpy as jnp
from jax import lax
from jax.experimental import pallas as pl
from jax.experimental.pallas import tpu as pltpu
from jax.experimental.pallas import tpu_sc as plsc

N = 10000
E = 320000
F_IN = 128
HID = 64
HEADS = 8
NCLS = 10
NG = 64

EA = E + N
K = 128
TILES = 16
CORES = 2
EAP = -(-EA // (CORES * TILES * K * 2)) * (CORES * TILES * K * 2)
NP = 10240
ROWS_PT = NP // TILES
K1 = 64
NP2 = 10112
RPT1 = NP2 // TILES
RT = 1000
RTM = 1024


def _sc_mesh():
    return plsc.VectorSubcoreMesh(
        core_axis_name="c", subcore_axis_name="s",
        num_cores=CORES, num_subcores=TILES)


def _pre1_body(x_ref, w_ref, as_ref, ad_ref, xl_ref, asd_ref, add_ref):
    xb = jnp.dot(x_ref[...], w_ref[...], preferred_element_type=jnp.float32)
    xl_ref[...] = xb
    a_s = (xb * as_ref[...]).reshape(RT, HEADS, HID).sum(-1)
    a_d = (xb * ad_ref[...]).reshape(RT, HEADS, HID).sum(-1)
    asd_ref[...] = jnp.concatenate([a_s, a_s], axis=1)
    add_ref[...] = jnp.concatenate([a_d, a_d], axis=1)


def _pre1(x, W1, as1, ad1):
    grid = (N // RT,)
    return pl.pallas_call(
        _pre1_body,
        grid=grid,
        in_specs=[
            pl.BlockSpec((RT, F_IN), lambda i: (i, 0)),
            pl.BlockSpec((F_IN, HEADS * HID), lambda i: (0, 0)),
            pl.BlockSpec((1, HEADS * HID), lambda i: (0, 0)),
            pl.BlockSpec((1, HEADS * HID), lambda i: (0, 0)),
        ],
        out_specs=[
            pl.BlockSpec((RT, HEADS * HID), lambda i: (i, 0)),
            pl.BlockSpec((RT, 16), lambda i: (i, 0)),
            pl.BlockSpec((RT, 16), lambda i: (i, 0)),
        ],
        out_shape=[
            jax.ShapeDtypeStruct((N, HEADS * HID), jnp.float32),
            jax.ShapeDtypeStruct((N, 16), jnp.float32),
            jax.ShapeDtypeStruct((N, 16), jnp.float32),
        ],
    )(x, W1, as1, ad1)


def _l1_body(src_hbm, dst_hbm, asd_hbm, add_hbm, xlc_hbm, z128_hbm, zd_hbm,
             num_out, den_out,
             sidxA, didxA, didxSA, sbufA, dbufA, wbufA, rowsA, srowsA,
             sidxB, didxB, didxSB, sbufB, dbufB, wbufB, rowsB, srowsB,
             num_sh, den_sh, semgA, semgB, semsA, semsB):
    c = lax.axis_index("c")
    s = lax.axis_index("s")
    r0 = s * RPT1
    ept = EAP // TILES
    nblk = ept // K1
    ebase = s * ept
    bufA = (sidxA, didxA, didxSA, sbufA, dbufA, wbufA, rowsA, srowsA,
            semgA, semsA)
    bufB = (sidxB, didxB, didxSB, sbufB, dbufB, wbufB, rowsB, srowsB,
            semgB, semsB)

    for cs in range(CORES):
        @pl.when(c == cs)
        def _core_branch(cs=cs):
            for ckl in range(2):
                ck = cs * 2 + ckl
                h0 = 2 * ck
                h1 = 2 * ck + 1
                first = (cs == 0 and ckl == 0)
                pltpu.sync_copy(z128_hbm.at[pl.ds(r0, RPT1)],
                                num_sh.at[pl.ds(r0, RPT1)])
                if first:
                    pltpu.sync_copy(zd_hbm.at[pl.ds(r0, RPT1)],
                                    den_sh.at[pl.ds(r0, RPT1)])
                plsc.subcore_barrier()

                def pregather(e0, buf, ck=ck):
                    sidx, didx, didxS, sbuf, dbuf, wbuf, rows, srows, \
                        semg, sems = buf
                    pltpu.sync_copy(src_hbm.at[pl.ds(e0, K1)], sidx)
                    pltpu.sync_copy(dst_hbm.at[pl.ds(e0, K1)], didx)
                    pltpu.async_copy(asd_hbm.at[sidx], sbuf, semg)
                    pltpu.async_copy(add_hbm.at[didx], dbuf, semg)
                    pltpu.async_copy(xlc_hbm.at[ck].at[sidx], rows, semg)

                def stage(j, buf, first=first, h0=h0, h1=h1):
                    sidx, didx, didxS, sbuf, dbuf, wbuf, rows, srows, \
                        semg, sems = buf
                    pltpu.make_async_copy(asd_hbm.at[sidx], sbuf, semg).wait()
                    pltpu.make_async_copy(add_hbm.at[didx], dbuf, semg).wait()
                    pltpu.make_async_copy(
                        xlc_hbm.at[0].at[sidx], rows, semg).wait()

                    @pl.when(j > 0)
                    def _drain_prev_scatter():
                        pltpu.make_async_copy(
                            srows, num_sh.at[didxS], sems).wait()
                        if first:
                            pltpu.make_async_copy(
                                wbuf, den_sh.at[didxS], sems).wait()

                    for t in range(K1 // 16):
                        didxS[pl.ds(t * 16, 16)] = didx[pl.ds(t * 16, 16)]

                    def edge8(k8, _):
                        for u in range(8):
                            k = k8 * 8 + u
                            al = sbuf[k, :] + dbuf[k, :]
                            al = jnp.maximum(al, 0.2 * al)
                            w = jnp.exp(al)
                            if first:
                                wbuf[k, :] = w
                            w0 = jnp.broadcast_to(w[h0], (16,))
                            w1 = jnp.broadcast_to(w[h1], (16,))
                            for t in range(4):
                                wv = w0 if t < 2 else w1
                                rb = rows[k, pl.ds(t * 32, 32)]
                                lo, hi = plsc.unpack(
                                    rb, format=plsc.PackFormat.INTERLEAVED)
                                srows[k, pl.ds(t * 32, 16)] = lo * wv
                                srows[k, pl.ds(t * 32 + 16, 16)] = hi * wv
                        return 0

                    lax.fori_loop(0, K1 // 8, edge8, 0)
                    pltpu.async_copy(srows, num_sh.at[didxS], sems, add=True)
                    if first:
                        pltpu.async_copy(wbuf, den_sh.at[didxS], sems,
                                         add=True)

                pregather(ebase, bufA)

                def body(j, _):
                    e0 = ebase + 2 * j * K1
                    pregather(e0 + K1, bufB)
                    stage(j, bufA)
                    pregather(e0 + 2 * K1, bufA)
                    stage(j, bufB)
                    return 0

                lax.fori_loop(0, nblk // 2, body, 0)
                pltpu.make_async_copy(asd_hbm.at[sidxA], sbufA, semgA).wait()
                pltpu.make_async_copy(add_hbm.at[didxA], dbufA, semgA).wait()
                pltpu.make_async_copy(
                    xlc_hbm.at[0].at[sidxA], rowsA, semgA).wait()
                pltpu.make_async_copy(srowsA, num_sh.at[didxSA], semsA).wait()
                pltpu.make_async_copy(srowsB, num_sh.at[didxSB], semsB).wait()
                if first:
                    pltpu.make_async_copy(
                        wbufA, den_sh.at[didxSA], semsA).wait()
                    pltpu.make_async_copy(
                        wbufB, den_sh.at[didxSB], semsB).wait()
                plsc.subcore_barrier()
                pltpu.sync_copy(num_sh.at[pl.ds(r0, RPT1)],
                                num_out.at[ck].at[pl.ds(r0, RPT1)])
                if first:
                    pltpu.sync_copy(den_sh.at[pl.ds(r0, RPT1)],
                                    den_out.at[pl.ds(r0, RPT1)])
                plsc.subcore_barrier()


def _l1_edge(src, dst, asd, add_, xlc, z128, zd):
    kfn = pl.kernel(
        _l1_body,
        out_type=[
            jax.ShapeDtypeStruct((4, NP, 128), jnp.float32),
            jax.ShapeDtypeStruct((NP, 16), jnp.float32),
        ],
        mesh=_sc_mesh(),
        compiler_params=pltpu.CompilerParams(
            use_tc_tiling_on_sc=False, needs_layout_passes=False),
        scratch_types=(
            [pltpu.VMEM((K1,), jnp.int32),
             pltpu.VMEM((K1,), jnp.int32),
             pltpu.VMEM((K1,), jnp.int32),
             pltpu.VMEM((K1, 16), jnp.float32),
             pltpu.VMEM((K1, 16), jnp.float32),
             pltpu.VMEM((K1, 16), jnp.float32),
             pltpu.VMEM((K1, 128), jnp.bfloat16),
             pltpu.VMEM((K1, 128), jnp.float32)] * 2
            + [pltpu.VMEM_SHARED((NP2, 128), jnp.float32),
               pltpu.VMEM_SHARED((NP2, 16), jnp.float32),
               pltpu.SemaphoreType.DMA,
               pltpu.SemaphoreType.DMA,
               pltpu.SemaphoreType.DMA,
               pltpu.SemaphoreType.DMA]),
    )
    return kfn(src, dst, asd, add_, xlc, z128, zd)


def _mid_body(num_ref, den_ref, b1_ref, w2_ref, as2_ref, ad2_ref,
              xl2_ref, asd2_ref, add2_ref):
    acc = jnp.zeros((RTM, HID), jnp.float32)
    for ck in range(4):
        nb = num_ref[ck]
        d0 = den_ref[:, 2 * ck]
        d1 = den_ref[:, 2 * ck + 1]
        div = jnp.concatenate(
            [jnp.broadcast_to(d0[:, None], (RTM, HID)),
             jnp.broadcast_to(d1[:, None], (RTM, HID))], axis=1)
        h = nb / (div + 1e-16) + b1_ref[0, 128 * ck:128 * ck + 128]
        h = jnp.maximum(h, 0.0)
        acc = acc + jnp.dot(h, w2_ref[128 * ck:128 * ck + 128, :],
                            preferred_element_type=jnp.float32)
    xl2_ref[...] = acc
    a_s = (acc * as2_ref[...]).sum(-1)
    a_d = (acc * ad2_ref[...]).sum(-1)
    asd2_ref[...] = jnp.broadcast_to(a_s[:, None], (RTM, 16))
    add2_ref[...] = jnp.broadcast_to(a_d[:, None], (RTM, 16))


def _mid(num1, den1, b1, W2, as2, ad2):
    grid = (N // RTM,)
    return pl.pallas_call(
        _mid_body,
        grid=grid,
        in_specs=[
            pl.BlockSpec((4, RTM, 128), lambda i: (0, i, 0)),
            pl.BlockSpec((RTM, 16), lambda i: (i, 0)),
            pl.BlockSpec((1, HEADS * HID), lambda i: (0, 0)),
            pl.BlockSpec((HEADS * HID, HID), lambda i: (0, 0)),
            pl.BlockSpec((1, HID), lambda i: (0, 0)),
            pl.BlockSpec((1, HID), lambda i: (0, 0)),
        ],
        out_specs=[
            pl.BlockSpec((RTM, HID), lambda i: (i, 0)),
            pl.BlockSpec((RTM, 16), lambda i: (i, 0)),
            pl.BlockSpec((RTM, 16), lambda i: (i, 0)),
        ],
        out_shape=[
            jax.ShapeDtypeStruct((NP, HID), jnp.float32),
            jax.ShapeDtypeStruct((NP, 16), jnp.float32),
            jax.ShapeDtypeStruct((NP, 16), jnp.float32),
        ],
    )(num1, den1, b1, W2, as2, ad2)


def _l2_body(src_hbm, dst_hbm, asd_hbm, add_hbm, xl2_hbm, z64_hbm, zd_hbm,
             num_out, den_out,
             sidxA, didxA, didxSA, sbufA, dbufA, wbufA, rowsA, srowsA,
             sidxB, didxB, didxSB, sbufB, dbufB, wbufB, rowsB, srowsB,
             num_sh, den_sh, semgA, semgB, semsA, semsB):
    c = lax.axis_index("c")
    s = lax.axis_index("s")
    r0 = s * ROWS_PT
    ept = EAP // (CORES * TILES)
    nblk = ept // K
    ebase = (c * TILES + s) * ept
    bufA = (sidxA, didxA, didxSA, sbufA, dbufA, wbufA, rowsA, srowsA,
            semgA, semsA)
    bufB = (sidxB, didxB, didxSB, sbufB, dbufB, wbufB, rowsB, srowsB,
            semgB, semsB)

    pltpu.sync_copy(z64_hbm.at[pl.ds(r0, ROWS_PT)],
                    num_sh.at[pl.ds(r0, ROWS_PT)])
    pltpu.sync_copy(zd_hbm.at[pl.ds(r0, ROWS_PT)],
                    den_sh.at[pl.ds(r0, ROWS_PT)])
    plsc.subcore_barrier()

    def pregather(e0, buf):
        sidx, didx, didxS, sbuf, dbuf, wbuf, rows, srows, semg, sems = buf
        pltpu.sync_copy(src_hbm.at[pl.ds(e0, K)], sidx)
        pltpu.sync_copy(dst_hbm.at[pl.ds(e0, K)], didx)
        pltpu.async_copy(asd_hbm.at[sidx], sbuf, semg)
        pltpu.async_copy(add_hbm.at[didx], dbuf, semg)
        pltpu.async_copy(xl2_hbm.at[sidx], rows, semg)

    def stage(j, buf):
        sidx, didx, didxS, sbuf, dbuf, wbuf, rows, srows, semg, sems = buf
        pltpu.make_async_copy(asd_hbm.at[sidx], sbuf, semg).wait()
        pltpu.make_async_copy(add_hbm.at[didx], dbuf, semg).wait()
        pltpu.make_async_copy(xl2_hbm.at[sidx], rows, semg).wait()

        @pl.when(j > 0)
        def _drain_prev_scatter():
            pltpu.make_async_copy(srows, num_sh.at[didxS], sems).wait()
            pltpu.make_async_copy(wbuf, den_sh.at[didxS], sems).wait()

        for t in range(K // 16):
            didxS[pl.ds(t * 16, 16)] = didx[pl.ds(t * 16, 16)]

        def edge8(k8, _):
            for u in range(8):
                k = k8 * 8 + u
                al = sbuf[k, :] + dbuf[k, :]
                al = jnp.maximum(al, 0.2 * al)
                w = jnp.exp(al)
                wbuf[k, :] = w
                w0 = jnp.broadcast_to(w[0], (16,))
                for t in range(2):
                    rb = rows[k, pl.ds(t * 32, 32)]
                    lo, hi = plsc.unpack(
                        rb, format=plsc.PackFormat.INTERLEAVED)
                    srows[k, pl.ds(t * 32, 16)] = lo * w0
                    srows[k, pl.ds(t * 32 + 16, 16)] = hi * w0
            return 0

        lax.fori_loop(0, K // 8, edge8, 0)
        pltpu.async_copy(srows, num_sh.at[didxS], sems, add=True)
        pltpu.async_copy(wbuf, den_sh.at[didxS], sems, add=True)

    pregather(ebase, bufA)

    def body(j, _):
        e0 = ebase + 2 * j * K
        pregather(e0 + K, bufB)
        stage(j, bufA)
        pregather(e0 + 2 * K, bufA)
        stage(j, bufB)
        return 0

    lax.fori_loop(0, nblk // 2, body, 0)
    pltpu.make_async_copy(asd_hbm.at[sidxA], sbufA, semgA).wait()
    pltpu.make_async_copy(add_hbm.at[didxA], dbufA, semgA).wait()
    pltpu.make_async_copy(xl2_hbm.at[sidxA], rowsA, semgA).wait()
    pltpu.make_async_copy(srowsA, num_sh.at[didxSA], semsA).wait()
    pltpu.make_async_copy(wbufA, den_sh.at[didxSA], semsA).wait()
    pltpu.make_async_copy(srowsB, num_sh.at[didxSB], semsB).wait()
    pltpu.make_async_copy(wbufB, den_sh.at[didxSB], semsB).wait()
    plsc.subcore_barrier()
    pltpu.sync_copy(num_sh.at[pl.ds(r0, ROWS_PT)],
                    num_out.at[c].at[pl.ds(r0, ROWS_PT)])
    pltpu.sync_copy(den_sh.at[pl.ds(r0, ROWS_PT)],
                    den_out.at[c].at[pl.ds(r0, ROWS_PT)])


def _l2_edge(src, dst, asd2, add2, xl2, z64, zd):
    kfn = pl.kernel(
        _l2_body,
        out_type=[
            jax.ShapeDtypeStruct((2, NP, HID), jnp.float32),
            jax.ShapeDtypeStruct((2, NP, 16), jnp.float32),
        ],
        mesh=_sc_mesh(),
        compiler_params=pltpu.CompilerParams(
            use_tc_tiling_on_sc=False, needs_layout_passes=False),
        scratch_types=(
            [pltpu.VMEM((K,), jnp.int32),
             pltpu.VMEM((K,), jnp.int32),
             pltpu.VMEM((K,), jnp.int32),
             pltpu.VMEM((K, 16), jnp.float32),
             pltpu.VMEM((K, 16), jnp.float32),
             pltpu.VMEM((K, 16), jnp.float32),
             pltpu.VMEM((K, HID), jnp.bfloat16),
             pltpu.VMEM((K, HID), jnp.float32)] * 2
            + [pltpu.VMEM_SHARED((NP, HID), jnp.float32),
               pltpu.VMEM_SHARED((NP, 16), jnp.float32),
               pltpu.SemaphoreType.DMA,
               pltpu.SemaphoreType.DMA,
               pltpu.SemaphoreType.DMA,
               pltpu.SemaphoreType.DMA]),
    )
    return kfn(src, dst, asd2, add2, xl2, z64, zd)


def _post_body(num_ref, den_ref, b2_ref, batch_ref, lw1_ref, lb1_ref,
               lw2_ref, lb2_ref, lw3_ref, lb3_ref, out_ref):
    num = num_ref[0] + num_ref[1]
    den = den_ref[0][:, 0] + den_ref[1][:, 0]
    h2 = jnp.maximum(num / (den[:, None] + 1e-16) + b2_ref[...], 0.0)
    valid = lax.broadcasted_iota(jnp.int32, (NP, 1), 0) < N
    h2 = jnp.where(valid, h2, 0.0)
    onehot = (batch_ref[...] ==
              lax.broadcasted_iota(jnp.int32, (NG, NP), 0)).astype(jnp.float32)
    sums = jnp.dot(onehot, h2, preferred_element_type=jnp.float32)
    cnt = jnp.sum(onehot, axis=1)
    g = sums / jnp.maximum(cnt, 1.0)[:, None]
    g = jnp.maximum(jnp.dot(g, lw1_ref[...],
                            preferred_element_type=jnp.float32) + lb1_ref[...], 0.0)
    g = jnp.maximum(jnp.dot(g, lw2_ref[...],
                            preferred_element_type=jnp.float32) + lb2_ref[...], 0.0)
    logits = jnp.dot(g, lw3_ref[...],
                     preferred_element_type=jnp.float32) + lb3_ref[...]
    m = jnp.max(logits, axis=-1, keepdims=True)
    lse = jnp.log(jnp.sum(jnp.exp(logits - m), axis=-1, keepdims=True)) + m
    out_ref[...] = logits - lse


def _post(num2, den2, b2, batch_i, lw1, lb1, lw2, lb2, lw3, lb3):
    return pl.pallas_call(
        _post_body,
        out_shape=jax.ShapeDtypeStruct((NG, NCLS), jnp.float32),
    )(num2, den2, b2, batch_i, lw1, lb1, lw2, lb2, lw3, lb3)


def kernel(x, edge_index, batch, W1, att_src1, att_dst1, b1,
           W2, att_src2, att_dst2, b2, lw1, lb1, lw2, lb2, lw3, lb3):
    loops = jnp.arange(N, dtype=jnp.int32)
    pad = jnp.full((EAP - EA + K,), N, jnp.int32)
    src = jnp.concatenate([edge_index[0].astype(jnp.int32), loops, pad])
    dst = jnp.concatenate([edge_index[1].astype(jnp.int32), loops, pad])

    as1 = att_src1.reshape(1, HEADS * HID)
    ad1 = att_dst1.reshape(1, HEADS * HID)
    xl, asd, add_ = _pre1(x, W1, as1, ad1)
    zrows16 = jnp.zeros((NP - N, 16), jnp.float32)
    asd = jnp.concatenate([asd, zrows16])
    add_ = jnp.concatenate([add_, zrows16])
    xlp = jnp.concatenate(
        [xl, jnp.zeros((NP - N, HEADS * HID), jnp.float32)])
    xli = xlp.reshape(NP, 16, 2, 16).transpose(0, 1, 3, 2).reshape(NP, 512)
    xlc = xli.astype(jnp.bfloat16).reshape(NP, 4, 128).transpose(1, 0, 2)

    z128 = jnp.zeros((NP, 128), jnp.float32)
    z64 = jnp.zeros((NP, HID), jnp.float32)
    zd = jnp.zeros((NP, 16), jnp.float32)
    num1, den1 = _l1_edge(src, dst, asd, add_, xlc, z128, zd)

    xl2, asd2, add2 = _mid(num1, den1, b1.reshape(1, HEADS * HID), W2,
                           att_src2.reshape(1, HID), att_dst2.reshape(1, HID))
    xl2i = xl2.reshape(NP, 2, 2, 16).transpose(0, 1, 3, 2).reshape(NP, HID)
    num2, den2 = _l2_edge(src, dst, asd2, add2,
                          xl2i.astype(jnp.bfloat16), z64, zd)

    return _post(num2, den2, b2.reshape(1, HID),
                 jnp.concatenate([batch.astype(jnp.int32), jnp.full((NP - N,), NG, jnp.int32)]).reshape(1, NP),
                 lw1, lb1.reshape(1, HID), lw2, lb2.reshape(1, HID),
                 lw3, lb3.reshape(1, NCLS))
